# baseline (speedup 1.0000x reference)
import jax
import jax.numpy as jnp
from jax.experimental import pallas as pl


def _row_block(m):
    for bm in (400, 200, 100, 40, 16, 8):
        if m % bm == 0:
            return bm
    return m


def _ln_relu(h, g, be):
    mu = jnp.mean(h, axis=-1, keepdims=True)
    d = h - mu
    var = jnp.mean(d * d, axis=-1, keepdims=True)
    return jnp.maximum(d * jax.lax.rsqrt(var + 1e-5) * g + be, 0.0)


def _feat_kernel(x_ref, w_ref, o_ref):
    o_ref[...] = jnp.dot(x_ref[...], w_ref[...],
                         preferred_element_type=jnp.float32)


def _pass1_kernel(adj_ref, y_ref, b_ref, g_ref, be_ref, w_ref, o_ref):
    h = jnp.dot(adj_ref[...], y_ref[...], preferred_element_type=jnp.float32)
    h = _ln_relu(h + b_ref[...], g_ref[...], be_ref[...])
    o_ref[...] = jnp.dot(h, w_ref[...], preferred_element_type=jnp.float32)


def _pass2_kernel(adj_ref, y_ref, b_ref, g_ref, be_ref, w_ref, h_ref, y3_ref):
    h = jnp.dot(adj_ref[...], y_ref[...], preferred_element_type=jnp.float32)
    h = _ln_relu(h + b_ref[...], g_ref[...], be_ref[...])
    h_ref[...] = h
    y3_ref[...] = jnp.dot(h, w_ref[...], preferred_element_type=jnp.float32)


def _pass3_kernel(adj_ref, y_ref, b_ref, g_ref, be_ref, hin_ref, o_ref):
    h = jnp.dot(adj_ref[...], y_ref[...], preferred_element_type=jnp.float32)
    h = _ln_relu(h + b_ref[...], g_ref[...], be_ref[...])
    o_ref[...] = h + hin_ref[...]


def _head_kernel(h_ref, wf1_ref, bf1_ref, wf2_ref, bf2_ref, o_ref):
    h = h_ref[...]
    m = jnp.mean(h, axis=0, keepdims=True)
    mx = jnp.max(h, axis=0, keepdims=True)
    gr = jnp.concatenate([m, mx], axis=1)
    out = jnp.maximum(
        jnp.dot(gr, wf1_ref[...], preferred_element_type=jnp.float32)
        + bf1_ref[...], 0.0)
    o_ref[...] = (jnp.dot(out, wf2_ref[...], preferred_element_type=jnp.float32)
                  + bf2_ref[...])


def _full(shape):
    return pl.BlockSpec(shape, lambda i: (0,) * len(shape))


def kernel(adj, features, W1, b1, g1, be1, W2, b2, g2, be2, W3, b3, g3, be3,
           Wf1, bf1, Wf2, bf2):
    m, n = adj.shape
    d_in = features.shape[1]
    c1 = W1.shape[1]
    c2 = W2.shape[1]
    c3 = W3.shape[1]
    bm = _row_block(m)
    grid = (m // bm,)

    b1r, g1r, be1r = b1[None, :], g1[None, :], be1[None, :]
    b2r, g2r, be2r = b2[None, :], g2[None, :], be2[None, :]
    b3r, g3r, be3r = b3[None, :], g3[None, :], be3[None, :]

    fb = _row_block(m)
    y1 = pl.pallas_call(
        _feat_kernel,
        grid=(m // fb,),
        in_specs=[pl.BlockSpec((fb, d_in), lambda i: (i, 0)),
                  _full((d_in, c1))],
        out_specs=pl.BlockSpec((fb, c1), lambda i: (i, 0)),
        out_shape=jax.ShapeDtypeStruct((m, c1), jnp.float32),
    )(features, W1)

    adj_spec = pl.BlockSpec((bm, n), lambda i: (i, 0))
    row_out = lambda c: pl.BlockSpec((bm, c), lambda i: (i, 0))

    y2 = pl.pallas_call(
        _pass1_kernel,
        grid=grid,
        in_specs=[adj_spec, _full((n, c1)), _full((1, c1)), _full((1, c1)),
                  _full((1, c1)), _full((c1, c2))],
        out_specs=row_out(c2),
        out_shape=jax.ShapeDtypeStruct((m, c2), jnp.float32),
    )(adj, y1, b1r, g1r, be1r, W2)

    h_in, y3 = pl.pallas_call(
        _pass2_kernel,
        grid=grid,
        in_specs=[adj_spec, _full((n, c2)), _full((1, c2)), _full((1, c2)),
                  _full((1, c2)), _full((c2, c3))],
        out_specs=[row_out(c2), row_out(c3)],
        out_shape=[jax.ShapeDtypeStruct((m, c2), jnp.float32),
                   jax.ShapeDtypeStruct((m, c3), jnp.float32)],
    )(adj, y2, b2r, g2r, be2r, W3)

    h3 = pl.pallas_call(
        _pass3_kernel,
        grid=grid,
        in_specs=[adj_spec, _full((n, c3)), _full((1, c3)), _full((1, c3)),
                  _full((1, c3)), row_out(c2)],
        out_specs=row_out(c3),
        out_shape=jax.ShapeDtypeStruct((m, c3), jnp.float32),
    )(adj, y3, b3r, g3r, be3r, h_in)

    nc = Wf2.shape[1]
    logits = pl.pallas_call(
        _head_kernel,
        out_shape=jax.ShapeDtypeStruct((1, nc), jnp.float32),
    )(h3, Wf1, bf1[None, :], Wf2, bf2[None, :])

    return logits


# speedup vs baseline: 1.0840x; 1.0840x over previous
import jax
import jax.numpy as jnp
from jax.experimental import pallas as pl


def _row_block(m):
    for bm in (400, 200, 100, 40, 16, 8):
        if m % bm == 0:
            return bm
    return m


def _ln_relu(h, g, be):
    mu = jnp.mean(h, axis=-1, keepdims=True)
    d = h - mu
    var = jnp.mean(d * d, axis=-1, keepdims=True)
    return jnp.maximum(d * jax.lax.rsqrt(var + 1e-5) * g + be, 0.0)


def _feat_kernel(x_ref, w_ref, o_ref):
    o_ref[...] = jnp.dot(x_ref[...], w_ref[...],
                         preferred_element_type=jnp.float32)


def _pass1_kernel(adj_ref, y_ref, b_ref, g_ref, be_ref, w_ref, o_ref, abf_ref):
    a = adj_ref[...]
    abf_ref[...] = a.astype(jnp.bfloat16)
    h = jnp.dot(a, y_ref[...], preferred_element_type=jnp.float32)
    h = _ln_relu(h + b_ref[...], g_ref[...], be_ref[...])
    o_ref[...] = jnp.dot(h, w_ref[...], preferred_element_type=jnp.float32)


def _pass2_kernel(adj_ref, y_ref, b_ref, g_ref, be_ref, w_ref, h_ref, y3_ref):
    h = jnp.dot(adj_ref[...].astype(jnp.float32), y_ref[...],
                preferred_element_type=jnp.float32)
    h = _ln_relu(h + b_ref[...], g_ref[...], be_ref[...])
    h_ref[...] = h
    y3_ref[...] = jnp.dot(h, w_ref[...], preferred_element_type=jnp.float32)


def _pass3_kernel(adj_ref, y_ref, b_ref, g_ref, be_ref, hin_ref, o_ref):
    h = jnp.dot(adj_ref[...].astype(jnp.float32), y_ref[...],
                preferred_element_type=jnp.float32)
    h = _ln_relu(h + b_ref[...], g_ref[...], be_ref[...])
    o_ref[...] = h + hin_ref[...]


def _head_kernel(h_ref, wf1_ref, bf1_ref, wf2_ref, bf2_ref, o_ref):
    h = h_ref[...]
    m = jnp.mean(h, axis=0, keepdims=True)
    mx = jnp.max(h, axis=0, keepdims=True)
    gr = jnp.concatenate([m, mx], axis=1)
    out = jnp.maximum(
        jnp.dot(gr, wf1_ref[...], preferred_element_type=jnp.float32)
        + bf1_ref[...], 0.0)
    o_ref[...] = (jnp.dot(out, wf2_ref[...], preferred_element_type=jnp.float32)
                  + bf2_ref[...])


def _full(shape):
    return pl.BlockSpec(shape, lambda i: (0,) * len(shape))


def kernel(adj, features, W1, b1, g1, be1, W2, b2, g2, be2, W3, b3, g3, be3,
           Wf1, bf1, Wf2, bf2):
    m, n = adj.shape
    d_in = features.shape[1]
    c1 = W1.shape[1]
    c2 = W2.shape[1]
    c3 = W3.shape[1]
    bm = _row_block(m)
    grid = (m // bm,)

    b1r, g1r, be1r = b1[None, :], g1[None, :], be1[None, :]
    b2r, g2r, be2r = b2[None, :], g2[None, :], be2[None, :]
    b3r, g3r, be3r = b3[None, :], g3[None, :], be3[None, :]

    fb = _row_block(m)
    y1 = pl.pallas_call(
        _feat_kernel,
        grid=(m // fb,),
        in_specs=[pl.BlockSpec((fb, d_in), lambda i: (i, 0)),
                  _full((d_in, c1))],
        out_specs=pl.BlockSpec((fb, c1), lambda i: (i, 0)),
        out_shape=jax.ShapeDtypeStruct((m, c1), jnp.float32),
    )(features, W1)

    adj_spec = pl.BlockSpec((bm, n), lambda i: (i, 0))
    row_out = lambda c: pl.BlockSpec((bm, c), lambda i: (i, 0))

    y2, adj_bf = pl.pallas_call(
        _pass1_kernel,
        grid=grid,
        in_specs=[adj_spec, _full((n, c1)), _full((1, c1)), _full((1, c1)),
                  _full((1, c1)), _full((c1, c2))],
        out_specs=[row_out(c2), pl.BlockSpec((bm, n), lambda i: (i, 0))],
        out_shape=[jax.ShapeDtypeStruct((m, c2), jnp.float32),
                   jax.ShapeDtypeStruct((m, n), jnp.bfloat16)],
    )(adj, y1, b1r, g1r, be1r, W2)

    h_in, y3 = pl.pallas_call(
        _pass2_kernel,
        grid=grid,
        in_specs=[adj_spec, _full((n, c2)), _full((1, c2)), _full((1, c2)),
                  _full((1, c2)), _full((c2, c3))],
        out_specs=[row_out(c2), row_out(c3)],
        out_shape=[jax.ShapeDtypeStruct((m, c2), jnp.float32),
                   jax.ShapeDtypeStruct((m, c3), jnp.float32)],
    )(adj_bf, y2, b2r, g2r, be2r, W3)

    h3 = pl.pallas_call(
        _pass3_kernel,
        grid=grid,
        in_specs=[adj_spec, _full((n, c3)), _full((1, c3)), _full((1, c3)),
                  _full((1, c3)), row_out(c2)],
        out_specs=row_out(c3),
        out_shape=jax.ShapeDtypeStruct((m, c3), jnp.float32),
    )(adj_bf, y3, b3r, g3r, be3r, h_in)

    nc = Wf2.shape[1]
    logits = pl.pallas_call(
        _head_kernel,
        out_shape=jax.ShapeDtypeStruct((1, nc), jnp.float32),
    )(h3, Wf1, bf1[None, :], Wf2, bf2[None, :])

    return logits


# speedup vs baseline: 1.2155x; 1.1212x over previous
import jax
import jax.numpy as jnp
from jax.experimental import pallas as pl


def _row_block(m):
    for bm in (400, 200, 100, 40, 16, 8):
        if m % bm == 0:
            return bm
    return m


def _ln_relu(h, g, be):
    mu = jnp.mean(h, axis=-1, keepdims=True)
    d = h - mu
    var = jnp.mean(d * d, axis=-1, keepdims=True)
    return jnp.maximum(d * jax.lax.rsqrt(var + 1e-5) * g + be, 0.0)


def _feat_kernel(x_ref, w_ref, o_ref):
    o_ref[...] = jnp.dot(x_ref[...], w_ref[...],
                         preferred_element_type=jnp.float32)


def _pass1_kernel(adj_ref, y_ref, b_ref, g_ref, be_ref, w_ref, o_ref,
                  aq_ref, sinv_ref):
    a = adj_ref[...]
    rowmax = jnp.max(a, axis=1, keepdims=True)
    aq_ref[...] = jnp.round(a * (127.0 / rowmax)).astype(jnp.int8)
    sinv_ref[...] = rowmax * (1.0 / 127.0)
    h = jnp.dot(a, y_ref[...], preferred_element_type=jnp.float32)
    h = _ln_relu(h + b_ref[...], g_ref[...], be_ref[...])
    o_ref[...] = jnp.dot(h, w_ref[...], preferred_element_type=jnp.float32)


def _pass2_kernel(adj_ref, y_ref, sinv_ref, b_ref, g_ref, be_ref, w_ref,
                  h_ref, y3_ref):
    h = jnp.dot(adj_ref[...].astype(jnp.bfloat16),
                y_ref[...].astype(jnp.bfloat16),
                preferred_element_type=jnp.float32)
    h = _ln_relu(h * sinv_ref[...] + b_ref[...], g_ref[...], be_ref[...])
    h_ref[...] = h
    y3_ref[...] = jnp.dot(h, w_ref[...], preferred_element_type=jnp.float32)


def _pass3_kernel(adj_ref, y_ref, sinv_ref, b_ref, g_ref, be_ref, hin_ref,
                  o_ref):
    h = jnp.dot(adj_ref[...].astype(jnp.bfloat16),
                y_ref[...].astype(jnp.bfloat16),
                preferred_element_type=jnp.float32)
    h = _ln_relu(h * sinv_ref[...] + b_ref[...], g_ref[...], be_ref[...])
    o_ref[...] = h + hin_ref[...]


def _head_kernel(h_ref, wf1_ref, bf1_ref, wf2_ref, bf2_ref, o_ref):
    h = h_ref[...]
    m = jnp.mean(h, axis=0, keepdims=True)
    mx = jnp.max(h, axis=0, keepdims=True)
    gr = jnp.concatenate([m, mx], axis=1)
    out = jnp.maximum(
        jnp.dot(gr, wf1_ref[...], preferred_element_type=jnp.float32)
        + bf1_ref[...], 0.0)
    o_ref[...] = (jnp.dot(out, wf2_ref[...], preferred_element_type=jnp.float32)
                  + bf2_ref[...])


def _full(shape):
    return pl.BlockSpec(shape, lambda i: (0,) * len(shape))


def kernel(adj, features, W1, b1, g1, be1, W2, b2, g2, be2, W3, b3, g3, be3,
           Wf1, bf1, Wf2, bf2):
    m, n = adj.shape
    d_in = features.shape[1]
    c1 = W1.shape[1]
    c2 = W2.shape[1]
    c3 = W3.shape[1]
    bm = _row_block(m)
    grid = (m // bm,)

    b1r, g1r, be1r = b1[None, :], g1[None, :], be1[None, :]
    b2r, g2r, be2r = b2[None, :], g2[None, :], be2[None, :]
    b3r, g3r, be3r = b3[None, :], g3[None, :], be3[None, :]

    fb = _row_block(m)
    y1 = pl.pallas_call(
        _feat_kernel,
        grid=(m // fb,),
        in_specs=[pl.BlockSpec((fb, d_in), lambda i: (i, 0)),
                  _full((d_in, c1))],
        out_specs=pl.BlockSpec((fb, c1), lambda i: (i, 0)),
        out_shape=jax.ShapeDtypeStruct((m, c1), jnp.float32),
    )(features, W1)

    adj_spec = pl.BlockSpec((bm, n), lambda i: (i, 0))
    row_out = lambda c: pl.BlockSpec((bm, c), lambda i: (i, 0))

    y2, adj_q, sinv = pl.pallas_call(
        _pass1_kernel,
        grid=grid,
        in_specs=[adj_spec, _full((n, c1)), _full((1, c1)), _full((1, c1)),
                  _full((1, c1)), _full((c1, c2))],
        out_specs=[row_out(c2), pl.BlockSpec((bm, n), lambda i: (i, 0)),
                   row_out(1)],
        out_shape=[jax.ShapeDtypeStruct((m, c2), jnp.float32),
                   jax.ShapeDtypeStruct((m, n), jnp.int8),
                   jax.ShapeDtypeStruct((m, 1), jnp.float32)],
    )(adj, y1, b1r, g1r, be1r, W2)

    h_in, y3 = pl.pallas_call(
        _pass2_kernel,
        grid=grid,
        in_specs=[adj_spec, _full((n, c2)), row_out(1), _full((1, c2)),
                  _full((1, c2)), _full((1, c2)), _full((c2, c3))],
        out_specs=[row_out(c2), row_out(c3)],
        out_shape=[jax.ShapeDtypeStruct((m, c2), jnp.float32),
                   jax.ShapeDtypeStruct((m, c3), jnp.float32)],
    )(adj_q, y2, sinv, b2r, g2r, be2r, W3)

    h3 = pl.pallas_call(
        _pass3_kernel,
        grid=grid,
        in_specs=[adj_spec, _full((n, c3)), row_out(1), _full((1, c3)),
                  _full((1, c3)), _full((1, c3)), row_out(c2)],
        out_specs=row_out(c3),
        out_shape=jax.ShapeDtypeStruct((m, c3), jnp.float32),
    )(adj_q, y3, sinv, b3r, g3r, be3r, h_in)

    nc = Wf2.shape[1]
    logits = pl.pallas_call(
        _head_kernel,
        out_shape=jax.ShapeDtypeStruct((1, nc), jnp.float32),
    )(h3, Wf1, bf1[None, :], Wf2, bf2[None, :])

    return logits
